# baseline (speedup 1.0000x reference)
import jax
import jax.numpy as jnp
from jax import lax
from jax.experimental import pallas as pl
from jax.experimental.pallas import tpu as pltpu

_SLOPE = 0.01
_SL = 128
_TN = 896
_GL = 128


def kernel(w0, b0, w1a, b1a, w1b, b1b, w2a, b2a, w2b, b2b, wr, br, ctx, res):
    B, Cc, H, W = ctx.shape
    Cr = res.shape[1]
    Cin0 = Cc + Cr
    C = w0.shape[-1]
    CR = wr.shape[-1]
    Hp = H + 2
    Lp = Hp * _SL
    NT = 2 * (-(-Lp // (2 * _TN)))
    NPAD = NT * _TN
    L = _GL + NPAD + _GL
    KW0 = 3 * Cin0
    KWC = 3 * C
    MW = 3 * C
    f32 = jnp.float32
    bf16 = jnp.bfloat16

    ctf = ctx.reshape(B, Cc, H * W)
    ref_ = res.reshape(B, Cr, H * W)

    def _wT(w, gs):
        cin, cout = w.shape[2], w.shape[3]
        wp = jnp.pad(w, ((0, 0), (0, 0), (0, 0), (0, gs - cout)))
        return jnp.transpose(wp, (1, 3, 0, 2)).reshape(3 * gs, 3 * cin)

    GR = 8
    w0m = _wT(w0, C).astype(bf16)
    wcm = jnp.stack([_wT(w, C) for w in (w1a, w1b, w2a, w2b)]).astype(bf16)
    wrm = _wT(wr, GR).astype(bf16)
    bm = jnp.stack([b.reshape(C, 1).astype(f32)
                    for b in (b0, b1a, b1b, b2a, b2b)])
    brm = jnp.pad(br, (0, GR - CR)).reshape(GR, 1).astype(f32)

    def body(ct_ref, rs_ref, w0_ref, wc_ref, wr_ref, bm_ref, br_ref,
             fout_ref, rout_ref, x_ref, feat_ref, rec_ref, act_ref, pb_ref):

        def lrelu(v):
            return jnp.where(v >= 0, v, _SLOPE * v)

        def interior(q0):
            l = q0 + lax.broadcasted_iota(jnp.int32, (1, _TN), 1)
            h = l >> 7
            w_ = l & 127
            return (h >= 1) & (h <= H) & (w_ >= 1) & (w_ <= W)

        def rolled_sum(p, gs):
            p0, p1, p2 = p[0:gs, :], p[gs:2 * gs, :], p[2 * gs:3 * gs, :]
            r0 = jnp.concatenate([p0[:, _TN - 1:], p0[:, :_TN - 1]], axis=1)
            r2 = jnp.concatenate([p2[:, 1:], p2[:, :1]], axis=1)
            return r0 + p1 + r2

        def tap_rhs(src_ref, cin, q0):
            v = src_ref[0:cin, pl.ds(_GL - _SL + q0, _TN + 2 * _SL)]
            return jnp.concatenate(
                [v[:, ky * _SL:ky * _SL + _TN] for ky in range(3)], axis=0)

        def conv_pass(src_ref, cin, w, bias, store, gs=C):
            def tile(q0):
                pv = jnp.dot(w, tap_rhs(src_ref, cin, q0),
                             preferred_element_type=f32)
                store(q0, rolled_sum(pv, gs) + bias)

            for j in range(NT):
                tile(j * _TN)

        for ref in (feat_ref, act_ref, pb_ref):
            ref[:, 0:_GL] = jnp.zeros((C, _GL), bf16)
            ref[:, _GL + NPAD:L] = jnp.zeros((C, L - _GL - NPAD), bf16)

        x_ref[:, 0:_GL + _SL] = jnp.zeros((Cin0, _GL + _SL), bf16)
        x_ref[:, _GL + _SL + H * _SL:L] = (
            jnp.zeros((Cin0, L - _GL - _SL - H * _SL), bf16))
        RB = 4 * W
        DB = 4 * _SL

        def pad4(i, carry):
            so = pl.multiple_of(i * RB, 128)
            do = pl.multiple_of(i * DB, 128)
            vc = ct_ref[:, pl.ds(so, RB)].astype(bf16)
            vr = rs_ref[:, pl.ds(so, RB)].astype(bf16)

            def blk(v):
                return jnp.concatenate(
                    [jnp.pad(v[:, j * W:(j + 1) * W], ((0, 0), (1, _SL - W - 1)))
                     for j in range(4)], axis=1)

            x_ref[0:Cc, pl.ds(_GL + _SL + do, DB)] = blk(vc)
            x_ref[Cc:Cin0, pl.ds(_GL + _SL + do, DB)] = blk(vr)
            return carry

        for j in range(H // 4):
            pad4(j, 0)

        def st_head(q0, y):
            y = jnp.where(interior(q0), y, 0.0)
            feat_ref[:, pl.ds(_GL + q0, _TN)] = y.astype(bf16)
            act_ref[:, pl.ds(_GL + q0, _TN)] = lrelu(y).astype(bf16)

        def st_mid(q0, y):
            y = jnp.where(interior(q0), lrelu(y), 0.0)
            pb_ref[:, pl.ds(_GL + q0, _TN)] = y.astype(bf16)

        def st_res(q0, y):
            y = (jnp.where(interior(q0), y, 0.0)
                 + feat_ref[:, pl.ds(_GL + q0, _TN)].astype(f32))
            feat_ref[:, pl.ds(_GL + q0, _TN)] = y.astype(bf16)
            act_ref[:, pl.ds(_GL + q0, _TN)] = lrelu(y).astype(bf16)

        def st_rec(q0, y):
            rec_ref[0:CR, pl.ds(_GL + q0, _TN)] = y[0:CR, :].astype(bf16)

        def out4(i, carry):
            do = pl.multiple_of(i * DB, 128)
            so = pl.multiple_of(i * RB, 128)
            vf = feat_ref[:, pl.ds(_GL + _SL + do, DB)]
            vr = rec_ref[0:CR, pl.ds(_GL + _SL + do, DB)]

            def blk(v):
                return jnp.concatenate(
                    [v[:, j * _SL + 1:j * _SL + 1 + W] for j in range(4)],
                    axis=1).astype(f32)

            fout_ref[:, pl.ds(so, RB)] = blk(vf)
            rout_ref[0:CR, pl.ds(so, RB)] = blk(vr)
            return carry

        conv_pass(x_ref, Cin0, w0_ref[...], bm_ref[0], st_head)
        conv_pass(act_ref, C, wc_ref[0], bm_ref[1], st_mid)
        conv_pass(pb_ref, C, wc_ref[1], bm_ref[2], st_res)
        conv_pass(act_ref, C, wc_ref[2], bm_ref[3], st_mid)
        conv_pass(pb_ref, C, wc_ref[3], bm_ref[4], st_res)
        conv_pass(feat_ref, C, wr_ref[...], br_ref[...], st_rec, gs=GR)
        for j in range(H // 4):
            out4(j, 0)

    feat, rec = pl.pallas_call(
        body,
        out_shape=(jax.ShapeDtypeStruct((B, C, H * W), f32),
                   jax.ShapeDtypeStruct((B, CR, H * W), f32)),
        grid=(B,),
        in_specs=[
            pl.BlockSpec((None, Cc, H * W), lambda b: (b, 0, 0)),
            pl.BlockSpec((None, Cr, H * W), lambda b: (b, 0, 0)),
            pl.BlockSpec((MW, KW0), lambda b: (0, 0)),
            pl.BlockSpec((4, MW, KWC), lambda b: (0, 0, 0)),
            pl.BlockSpec((3 * GR, KWC), lambda b: (0, 0)),
            pl.BlockSpec((5, C, 1), lambda b: (0, 0, 0)),
            pl.BlockSpec((GR, 1), lambda b: (0, 0)),
        ],
        out_specs=(pl.BlockSpec((None, C, H * W), lambda b: (b, 0, 0)),
                   pl.BlockSpec((None, CR, H * W), lambda b: (b, 0, 0))),
        scratch_shapes=[pltpu.VMEM((Cin0, L), bf16),
                        pltpu.VMEM((C, L), bf16),
                        pltpu.VMEM((8, L), bf16),
                        pltpu.VMEM((C, L), bf16),
                        pltpu.VMEM((C, L), bf16)],
        compiler_params=pltpu.CompilerParams(
            dimension_semantics=("parallel",),
            vmem_limit_bytes=100 << 20),
    )(ctf, ref_, w0m, wcm, wrm, bm, brm)

    return (feat.reshape(B, C, H, W).astype(ctx.dtype),
            rec.reshape(B, CR, H, W).astype(ctx.dtype))
